# baseline (speedup 1.0000x reference)
import jax
import jax.numpy as jnp
from jax.experimental import pallas as pl
from jax.experimental.pallas import tpu as pltpu

NUM_GRAPHS = 128
HIDDEN = 256
BLOCK = 2048


def _pool_kernel(firsts, lasts, x_ref, seg_ref, segr_ref, wt_ref, b_ref,
                 o_ref, mx_ref, sm_ref, ct_ref):
    i = pl.program_id(0)
    nb = pl.num_programs(0)

    @pl.when(i == 0)
    def _():
        mx_ref[...] = jnp.full_like(mx_ref, -jnp.inf)
        sm_ref[...] = jnp.zeros_like(sm_ref)
        ct_ref[...] = jnp.zeros_like(ct_ref)

    x = x_ref[...]
    seg = seg_ref[...]
    first = firsts[i]
    last = lasts[i]

    def body(s, carry):
        m = seg == s
        xm = jnp.where(m, x, -jnp.inf)
        xs = jnp.where(m, x, 0.0)
        bmax = jnp.max(xm, axis=0, keepdims=True)
        bsum = jnp.sum(xs, axis=0, keepdims=True)
        bcnt = jnp.sum(m.astype(jnp.float32), axis=0, keepdims=True)
        mx_ref[pl.ds(s, 1), :] = jnp.maximum(mx_ref[pl.ds(s, 1), :], bmax)
        sm_ref[pl.ds(s, 1), :] = sm_ref[pl.ds(s, 1), :] + bsum
        ct_ref[pl.ds(s, 1), :] = ct_ref[pl.ds(s, 1), :] + jnp.broadcast_to(
            bcnt, (1, HIDDEN))
        return carry

    jax.lax.fori_loop(first, last + 1, body, 0)

    @pl.when(i == nb - 1)
    def _():
        mean = sm_ref[...] / jnp.maximum(ct_ref[...], 1.0)
        comb = jnp.concatenate([mx_ref[...], mean], axis=1)
        o_ref[...] = jax.lax.dot_general(
            comb, wt_ref[...], (((1,), (0,)), ((), ())),
            preferred_element_type=jnp.float32) + b_ref[...]


@jax.jit
def kernel(x, batch, W, b):
    n, h = x.shape
    batch = batch.astype(jnp.int32)
    nb = pl.cdiv(n, BLOCK)
    npad = nb * BLOCK
    x = jnp.pad(x, ((0, npad - n), (0, 0)))
    segp = jnp.pad(batch, (0, npad - n), constant_values=NUM_GRAPHS)
    firsts = segp[::BLOCK]
    lasts = jnp.minimum(segp[BLOCK - 1::BLOCK], NUM_GRAPHS - 1)
    seg2d = segp.reshape(npad, 1)
    seg3d = segp.reshape(nb, 1, BLOCK)
    wt = W.T
    b2 = b.reshape(1, h)

    out = pl.pallas_call(
        _pool_kernel,
        grid_spec=pltpu.PrefetchScalarGridSpec(
            num_scalar_prefetch=2,
            grid=(nb,),
            in_specs=[
                pl.BlockSpec((BLOCK, h), lambda i, *_: (i, 0)),
                pl.BlockSpec((BLOCK, 1), lambda i, *_: (i, 0)),
                pl.BlockSpec((1, 1, BLOCK), lambda i, *_: (i, 0, 0)),
                pl.BlockSpec((2 * h, h), lambda i, *_: (0, 0)),
                pl.BlockSpec((1, h), lambda i, *_: (0, 0)),
            ],
            out_specs=pl.BlockSpec((NUM_GRAPHS, h), lambda i, *_: (0, 0)),
            scratch_shapes=[
                pltpu.VMEM((NUM_GRAPHS, h), jnp.float32),
                pltpu.VMEM((NUM_GRAPHS, h), jnp.float32),
                pltpu.VMEM((NUM_GRAPHS, h), jnp.float32),
            ],
        ),
        out_shape=jax.ShapeDtypeStruct((NUM_GRAPHS, h), jnp.float32),
    )(firsts, lasts, x, seg2d, seg3d, wt, b2)
    return out


# speedup vs baseline: 1.2056x; 1.2056x over previous
import jax
import jax.numpy as jnp
from jax.experimental import pallas as pl
from jax.experimental.pallas import tpu as pltpu

NUM_GRAPHS = 128
HIDDEN = 256
BLOCK = 1024


def _pool_kernel(firsts, lasts, x_ref, seg_ref, segr_ref, wt_ref, b_ref,
                 o_ref, mx_ref, sm_ref, ct_ref):
    i = pl.program_id(0)
    nb = pl.num_programs(0)

    @pl.when(i == 0)
    def _():
        mx_ref[...] = jnp.full_like(mx_ref, -jnp.inf)
        sm_ref[...] = jnp.zeros_like(sm_ref)
        ct_ref[...] = jnp.zeros_like(ct_ref)

    first = firsts[i]
    last = lasts[i]

    def body(s, carry):
        acc_mx = jnp.full((8, HIDDEN), -jnp.inf, dtype=jnp.float32)
        acc_sm = jnp.zeros((8, HIDDEN), dtype=jnp.float32)
        acc_ct = jnp.zeros((8, 1), dtype=jnp.float32)
        for k in range(BLOCK // 64):
            xk = x_ref[k * 64:(k + 1) * 64, :]
            mk = seg_ref[k * 64:(k + 1) * 64, :] == s
            xm = jnp.where(mk, xk, -jnp.inf).reshape(8, 8, HIDDEN)
            xs = jnp.where(mk, xk, 0.0).reshape(8, 8, HIDDEN)
            acc_mx = jnp.maximum(acc_mx, jnp.max(xm, axis=0))
            acc_sm = acc_sm + jnp.sum(xs, axis=0)
            acc_ct = acc_ct + jnp.sum(
                mk.astype(jnp.float32).reshape(8, 8, 1), axis=0)
        bmax = jnp.max(acc_mx, axis=0, keepdims=True)
        bsum = jnp.sum(acc_sm, axis=0, keepdims=True)
        bcnt = jnp.sum(acc_ct, axis=0, keepdims=True)
        mx_ref[pl.ds(s, 1), :] = jnp.maximum(mx_ref[pl.ds(s, 1), :], bmax)
        sm_ref[pl.ds(s, 1), :] = sm_ref[pl.ds(s, 1), :] + bsum
        ct_ref[pl.ds(s, 1), :] = ct_ref[pl.ds(s, 1), :] + jnp.broadcast_to(
            bcnt, (1, HIDDEN))
        return carry

    jax.lax.fori_loop(first, last + 1, body, 0)

    @pl.when(i == nb - 1)
    def _():
        mean = sm_ref[...] / jnp.maximum(ct_ref[...], 1.0)
        comb = jnp.concatenate([mx_ref[...], mean], axis=1)
        o_ref[...] = jax.lax.dot_general(
            comb, wt_ref[...], (((1,), (0,)), ((), ())),
            preferred_element_type=jnp.float32) + b_ref[...]


@jax.jit
def kernel(x, batch, W, b):
    n, h = x.shape
    batch = batch.astype(jnp.int32)
    nb = pl.cdiv(n, BLOCK)
    npad = nb * BLOCK
    x = jnp.pad(x, ((0, npad - n), (0, 0)))
    segp = jnp.pad(batch, (0, npad - n), constant_values=NUM_GRAPHS)
    firsts = segp[::BLOCK]
    lasts = jnp.minimum(segp[BLOCK - 1::BLOCK], NUM_GRAPHS - 1)
    seg2d = segp.reshape(npad, 1)
    seg3d = segp.reshape(nb, 1, BLOCK)
    wt = W.T
    b2 = b.reshape(1, h)

    out = pl.pallas_call(
        _pool_kernel,
        grid_spec=pltpu.PrefetchScalarGridSpec(
            num_scalar_prefetch=2,
            grid=(nb,),
            in_specs=[
                pl.BlockSpec((BLOCK, h), lambda i, *_: (i, 0)),
                pl.BlockSpec((BLOCK, 1), lambda i, *_: (i, 0)),
                pl.BlockSpec((1, 1, BLOCK), lambda i, *_: (i, 0, 0)),
                pl.BlockSpec((2 * h, h), lambda i, *_: (0, 0)),
                pl.BlockSpec((1, h), lambda i, *_: (0, 0)),
            ],
            out_specs=pl.BlockSpec((NUM_GRAPHS, h), lambda i, *_: (0, 0)),
            scratch_shapes=[
                pltpu.VMEM((NUM_GRAPHS, h), jnp.float32),
                pltpu.VMEM((NUM_GRAPHS, h), jnp.float32),
                pltpu.VMEM((NUM_GRAPHS, h), jnp.float32),
            ],
        ),
        out_shape=jax.ShapeDtypeStruct((NUM_GRAPHS, h), jnp.float32),
    )(firsts, lasts, x, seg2d, seg3d, wt, b2)
    return out
